# baseline (speedup 1.0000x reference)
import functools

import jax
import jax.numpy as jnp
import numpy as np
from jax import lax
from jax.experimental import pallas as pl
from jax.experimental.pallas import tpu as pltpu
from jax.experimental.pallas import tpu_sc as plsc

_K = 512
_NC = 2
_NS = 16
_L = 16
_ROW = 1024
_NCHUNK = 4
_CH = 128 // _NCHUNK
_BLOCKS_PER_W = 64 // (_NC * _NS)
_CAPL = 3072

_SIGN = np.int32(-2147483648)


def _ukey(x):
    ui = lax.bitcast_convert_type(x, jnp.int32)
    m = lax.shift_right_arithmetic(ui, np.int32(31))
    return lax.bitcast_convert_type(ui ^ (m | _SIGN), jnp.uint32)


def _sc_body(s_hbm, out_hbm, buf0, buf1, hist_v, cand_v,
             si0, si1, so0, so1):
    lane = lax.iota(jnp.int32, _L)
    ones = jnp.ones((_L,), jnp.int32)
    zeros16 = jnp.zeros((_L,), jnp.int32)
    one_f = jnp.ones((_L,), jnp.float32)
    zero_f = jnp.zeros((_L,), jnp.float32)
    partner = lane ^ 8

    bufs = (buf0, buf1)
    in_sems = (si0, si1)
    out_sems = (so0, so1)

    wid = lax.axis_index("s") * _NC + lax.axis_index("c")
    blk_b = [wid * _BLOCKS_PER_W + blk for blk in range(_BLOCKS_PER_W)]

    def zero_hist():
        def zero_it(i, _):
            for u in range(8):
                hist_v[pl.ds(i * (_L * 8) + u * _L, _L)] = zeros16
            return 0

        lax.fori_loop(0, 256 // 8, zero_it, 0)

    def hist_sweep(data_v):
        def hist_row(i, _):
            def hist_it(jj, _2):
                for u in range(8):
                    x = data_v[i, pl.ds((jj * 8 + u) * _L, _L)]
                    uk = _ukey(x)
                    bk = lax.shift_right_logical(uk, np.uint32(24))
                    addr = lax.bitcast_convert_type(
                        bk, jnp.int32) * 16 + lane
                    plsc.addupdate_scatter(hist_v, [addr], ones)
                return 0

            lax.fori_loop(0, _ROW // (_L * 8), hist_it, 0)
            return 0

        lax.fori_loop(0, _CH, hist_row, 0)

    def compact_sweep(data_v, b1, off):
        def comp_row(i, off_c):
            def comp_it(jj, off_c2):
                for u in range(8):
                    x = data_v[i, pl.ds((jj * 8 + u) * _L, _L)]
                    uk = _ukey(x)
                    pred = lax.shift_right_logical(
                        uk, np.uint32(24)) == b1
                    addr = jnp.minimum(
                        off_c2, np.int32(_CAPL - 1)) * 16 + lane
                    plsc.store_scatter(
                        cand_v, [addr],
                        lax.bitcast_convert_type(uk, jnp.int32),
                        mask=pred)
                    off_c2 = off_c2 + pred.astype(jnp.int32)
                return off_c2

            return lax.fori_loop(0, _ROW // (_L * 8), comp_it, off_c)

        return lax.fori_loop(0, _CH, comp_row, off)

    def cand_pass(p, prefix, off, nrow4):
        sh_bk = 24 - 8 * p
        sh_pr = 32 - 8 * p

        def cbody(j, _):
            for u in range(4):
                r = j * 4 + u
                v = cand_v[pl.ds(r * _L, _L)]
                uk = lax.bitcast_convert_type(v, jnp.uint32)
                valid = jnp.broadcast_to(r, (_L,)).astype(jnp.int32) < off
                if p == 1:
                    keep = valid
                else:
                    keep = jnp.logical_and(
                        valid,
                        lax.shift_right_logical(
                            uk, np.uint32(sh_pr)) == prefix)
                bk = lax.shift_right_logical(
                    uk, np.uint32(sh_bk)) & np.uint32(0xFF)
                addr = lax.bitcast_convert_type(bk, jnp.int32) * 16 + lane
                plsc.addupdate_scatter(hist_v, [addr], ones, mask=keep)
            return 0

        lax.fori_loop(0, nrow4, cbody, 0)

    def scan_hist(prefix, kk):
        def scan_it(t, carry):
            cum, sel, above, found = carry
            bucket = 255 - t
            v = plsc.load_gather(hist_v, [bucket * 16 + lane])
            vsw = plsc.load_gather(hist_v, [bucket * 16 + partner])
            cum_new = cum + v + vsw
            newly = jnp.logical_and(jnp.logical_not(found), cum_new >= kk)
            bvec = jnp.broadcast_to(bucket, (_L,)).astype(jnp.int32)
            sel = jnp.where(newly, bvec, sel)
            above = jnp.where(newly, cum, above)
            return cum_new, sel, above, jnp.logical_or(found, newly)

        z = jnp.zeros((_L,), jnp.int32)
        _, sel, above, _ = lax.fori_loop(
            0, 256, scan_it, (z, z, z, jnp.zeros((_L,), jnp.bool_)))
        kk = kk - above
        prefix = (prefix << np.uint32(8)) | lax.bitcast_convert_type(
            sel, jnp.uint32)
        return prefix, kk

    def out_sweep(data_v, kth):
        def out_row(i, _):
            def out_it(jj, _2):
                for u in range(8):
                    sl = pl.ds((jj * 8 + u) * _L, _L)
                    uk = _ukey(data_v[i, sl])
                    data_v[i, sl] = jnp.where(uk >= kth, one_f, zero_f)
                return 0

            lax.fori_loop(0, _ROW // (_L * 8), out_it, 0)
            return 0

        lax.fori_loop(0, _CH, out_row, 0)

    sweeps = []
    for blk in range(_BLOCKS_PER_W):
        for c in range(_NCHUNK):
            sweeps.append(("hist", blk, c))
        for c in range(_NCHUNK):
            sweeps.append(("compact", blk, c))
        for c in range(_NCHUNK):
            sweeps.append(("out", blk, c))

    def src_slice(i):
        _, blk, c = sweeps[i]
        return (blk_b[blk], pl.ds(c * _CH, _CH))

    copies = {}
    out_pending = [None, None]

    def issue_in(i):
        if i >= len(sweeps):
            return
        nb = i % 2
        if out_pending[nb] is not None:
            out_pending[nb].wait()
            out_pending[nb] = None
        b, sl = src_slice(i)
        cp = pltpu.make_async_copy(s_hbm.at[b, sl], bufs[nb], in_sems[nb])
        cp.start()
        copies[i] = cp

    zero_hist()
    issue_in(0)
    issue_in(1)

    prefix = jnp.zeros((_L,), jnp.uint32)
    kk = jnp.full((_L,), _K, jnp.int32)
    off = jnp.zeros((_L,), jnp.int32)
    kth = None

    for i, (kind, blk, c) in enumerate(sweeps):
        nb = i % 2
        copies.pop(i).wait()
        if kind == "hist":
            if c == 0:
                prefix = jnp.zeros((_L,), jnp.uint32)
                kk = jnp.full((_L,), _K, jnp.int32)
                off = jnp.zeros((_L,), jnp.int32)
            hist_sweep(bufs[nb])
            issue_in(i + 2)
            if c == _NCHUNK - 1:
                prefix, kk = scan_hist(prefix, kk)
        elif kind == "compact":
            off = compact_sweep(bufs[nb], prefix, off)
            issue_in(i + 2)
            if c == _NCHUNK - 1:
                max_off = lax.reduce_max(off, (0,))
                nrow4 = lax.div(max_off + 3, np.int32(4))
                for p in (1, 2, 3):
                    zero_hist()
                    cand_pass(p, prefix, off, nrow4)
                    prefix, kk = scan_hist(prefix, kk)
                kth = prefix
                zero_hist()
        else:
            out_sweep(bufs[nb], kth)
            b, sl = src_slice(i)
            ocp = pltpu.make_async_copy(
                bufs[nb], out_hbm.at[b, sl], out_sems[nb])
            ocp.start()
            out_pending[nb] = ocp
            issue_in(i + 2)

    for nb in (0, 1):
        if out_pending[nb] is not None:
            out_pending[nb].wait()


@jax.jit
def kernel(scores):
    bsz, nmax, _, ensemble = scores.shape
    s3 = scores.reshape(bsz, nmax, nmax * ensemble)
    run = functools.partial(
        pl.kernel,
        mesh=plsc.VectorSubcoreMesh(core_axis_name="c",
                                    subcore_axis_name="s"),
        out_type=jax.ShapeDtypeStruct(s3.shape, jnp.float32),
        compiler_params=pltpu.CompilerParams(
            needs_layout_passes=False, use_tc_tiling_on_sc=False),
        scratch_types=[
            pltpu.VMEM((_CH, _ROW), jnp.float32),
            pltpu.VMEM((_CH, _ROW), jnp.float32),
            pltpu.VMEM((256 * _L,), jnp.int32),
            pltpu.VMEM((_CAPL * _L,), jnp.int32),
            pltpu.SemaphoreType.DMA,
            pltpu.SemaphoreType.DMA,
            pltpu.SemaphoreType.DMA,
            pltpu.SemaphoreType.DMA,
        ],
    )(_sc_body)
    out3 = run(s3)
    return out3.reshape(bsz, nmax, nmax, ensemble)


# speedup vs baseline: 1.1520x; 1.1520x over previous
import functools

import jax
import jax.numpy as jnp
import numpy as np
from jax import lax
from jax.experimental import pallas as pl
from jax.experimental.pallas import tpu as pltpu
from jax.experimental.pallas import tpu_sc as plsc

_K = 512
_NC = 2
_NS = 16
_L = 16
_ROW = 1024
_NCHUNK = 4
_CH = 128 // _NCHUNK
_BLOCKS_PER_W = 64 // (_NC * _NS)
_CAPL = 3072

_SIGN = np.int32(-2147483648)


def _ukey(x):
    ui = lax.bitcast_convert_type(x, jnp.int32)
    m = lax.shift_right_arithmetic(ui, np.int32(31))
    return lax.bitcast_convert_type(ui ^ (m | _SIGN), jnp.uint32)


def _sc_body(s_hbm, kth_hbm, buf0, buf1, hist_v, cand_v, kth_v,
             si0, si1):
    lane = lax.iota(jnp.int32, _L)
    ones = jnp.ones((_L,), jnp.int32)
    zeros16 = jnp.zeros((_L,), jnp.int32)
    one_f = jnp.ones((_L,), jnp.float32)
    zero_f = jnp.zeros((_L,), jnp.float32)
    partner = lane ^ 8

    bufs = (buf0, buf1)
    in_sems = (si0, si1)

    wid = lax.axis_index("s") * _NC + lax.axis_index("c")
    blk_b = [wid * _BLOCKS_PER_W + blk for blk in range(_BLOCKS_PER_W)]

    def zero_hist():
        def zero_it(i, _):
            for u in range(8):
                hist_v[pl.ds(i * (_L * 8) + u * _L, _L)] = zeros16
            return 0

        lax.fori_loop(0, 256 // 8, zero_it, 0)

    def hist_sweep(data_v):
        def hist_row(i, _):
            def hist_it(jj, _2):
                for u in range(8):
                    x = data_v[i, pl.ds((jj * 8 + u) * _L, _L)]
                    uk = _ukey(x)
                    bk = lax.shift_right_logical(uk, np.uint32(24))
                    addr = lax.bitcast_convert_type(
                        bk, jnp.int32) * 16 + lane
                    plsc.addupdate_scatter(hist_v, [addr], ones)
                return 0

            lax.fori_loop(0, _ROW // (_L * 8), hist_it, 0)
            return 0

        lax.fori_loop(0, _CH, hist_row, 0)

    def compact_sweep(data_v, b1, off):
        def comp_row(i, off_c):
            def comp_it(jj, off_c2):
                for u in range(8):
                    x = data_v[i, pl.ds((jj * 8 + u) * _L, _L)]
                    uk = _ukey(x)
                    pred = lax.shift_right_logical(
                        uk, np.uint32(24)) == b1
                    addr = jnp.minimum(
                        off_c2, np.int32(_CAPL - 1)) * 16 + lane
                    plsc.store_scatter(
                        cand_v, [addr],
                        lax.bitcast_convert_type(uk, jnp.int32),
                        mask=pred)
                    off_c2 = off_c2 + pred.astype(jnp.int32)
                return off_c2

            return lax.fori_loop(0, _ROW // (_L * 8), comp_it, off_c)

        return lax.fori_loop(0, _CH, comp_row, off)

    def cand_pass(p, prefix, off, nrow4):
        sh_bk = 24 - 8 * p
        sh_pr = 32 - 8 * p

        def cbody(j, _):
            for u in range(4):
                r = j * 4 + u
                v = cand_v[pl.ds(r * _L, _L)]
                uk = lax.bitcast_convert_type(v, jnp.uint32)
                valid = jnp.broadcast_to(r, (_L,)).astype(jnp.int32) < off
                if p == 1:
                    keep = valid
                else:
                    keep = jnp.logical_and(
                        valid,
                        lax.shift_right_logical(
                            uk, np.uint32(sh_pr)) == prefix)
                bk = lax.shift_right_logical(
                    uk, np.uint32(sh_bk)) & np.uint32(0xFF)
                addr = lax.bitcast_convert_type(bk, jnp.int32) * 16 + lane
                plsc.addupdate_scatter(hist_v, [addr], ones, mask=keep)
            return 0

        lax.fori_loop(0, nrow4, cbody, 0)

    def scan_hist(prefix, kk):
        def scan_it(t, carry):
            cum, sel, above, found = carry
            bucket = 255 - t
            v = plsc.load_gather(hist_v, [bucket * 16 + lane])
            vsw = plsc.load_gather(hist_v, [bucket * 16 + partner])
            cum_new = cum + v + vsw
            newly = jnp.logical_and(jnp.logical_not(found), cum_new >= kk)
            bvec = jnp.broadcast_to(bucket, (_L,)).astype(jnp.int32)
            sel = jnp.where(newly, bvec, sel)
            above = jnp.where(newly, cum, above)
            return cum_new, sel, above, jnp.logical_or(found, newly)

        z = jnp.zeros((_L,), jnp.int32)
        _, sel, above, _ = lax.fori_loop(
            0, 256, scan_it, (z, z, z, jnp.zeros((_L,), jnp.bool_)))
        kk = kk - above
        prefix = (prefix << np.uint32(8)) | lax.bitcast_convert_type(
            sel, jnp.uint32)
        return prefix, kk

    sweeps = []
    for blk in range(_BLOCKS_PER_W):
        for c in range(_NCHUNK):
            sweeps.append(("hist", blk, c))
        for c in range(_NCHUNK):
            sweeps.append(("compact", blk, c))

    def src_slice(i):
        _, blk, c = sweeps[i]
        return (blk_b[blk], pl.ds(c * _CH, _CH))

    copies = {}

    def issue_in(i):
        if i >= len(sweeps):
            return
        nb = i % 2
        b, sl = src_slice(i)
        cp = pltpu.make_async_copy(s_hbm.at[b, sl], bufs[nb], in_sems[nb])
        cp.start()
        copies[i] = cp

    zero_hist()
    issue_in(0)
    issue_in(1)

    prefix = jnp.zeros((_L,), jnp.uint32)
    kk = jnp.full((_L,), _K, jnp.int32)
    off = jnp.zeros((_L,), jnp.int32)

    for i, (kind, blk, c) in enumerate(sweeps):
        nb = i % 2
        copies.pop(i).wait()
        if kind == "hist":
            if c == 0:
                prefix = jnp.zeros((_L,), jnp.uint32)
                kk = jnp.full((_L,), _K, jnp.int32)
                off = jnp.zeros((_L,), jnp.int32)
            hist_sweep(bufs[nb])
            issue_in(i + 2)
            if c == _NCHUNK - 1:
                prefix, kk = scan_hist(prefix, kk)
        elif kind == "compact":
            off = compact_sweep(bufs[nb], prefix, off)
            issue_in(i + 2)
            if c == _NCHUNK - 1:
                max_off = lax.reduce_max(off, (0,))
                nrow4 = lax.div(max_off + 3, np.int32(4))
                for p in (1, 2, 3):
                    zero_hist()
                    cand_pass(p, prefix, off, nrow4)
                    prefix, kk = scan_hist(prefix, kk)
                zero_hist()
                kth_v[pl.ds(0, _L)] = lax.bitcast_convert_type(
                    prefix, jnp.int32)
                pltpu.sync_copy(kth_v.at[pl.ds(0, 8)],
                                kth_hbm.at[blk_b[blk]])


def _mask_body(kth_ref, x_ref, o_ref):
    x = x_ref[...]
    ui = lax.bitcast_convert_type(x, jnp.int32)
    m = lax.shift_right_arithmetic(ui, np.int32(31))
    uk = lax.bitcast_convert_type(ui ^ (m | _SIGN), jnp.uint32)
    idx8 = lax.broadcasted_iota(jnp.int32, (1, 1, _ROW), 2) % 8
    t = jnp.zeros((1, 1, _ROW), jnp.uint32)
    for e in range(8):
        te = lax.bitcast_convert_type(kth_ref[0, 0, e], jnp.uint32)
        t = jnp.where(idx8 == e, te, t)
    o_ref[...] = jnp.where(uk >= t, np.float32(1.0), np.float32(0.0))


@jax.jit
def kernel(scores):
    bsz, nmax, _, ensemble = scores.shape
    s3 = scores.reshape(bsz, nmax, nmax * ensemble)
    run = functools.partial(
        pl.kernel,
        mesh=plsc.VectorSubcoreMesh(core_axis_name="c",
                                    subcore_axis_name="s"),
        out_type=jax.ShapeDtypeStruct((bsz, ensemble), jnp.int32),
        compiler_params=pltpu.CompilerParams(
            needs_layout_passes=False, use_tc_tiling_on_sc=False),
        scratch_types=[
            pltpu.VMEM((_CH, _ROW), jnp.float32),
            pltpu.VMEM((_CH, _ROW), jnp.float32),
            pltpu.VMEM((256 * _L,), jnp.int32),
            pltpu.VMEM((_CAPL * _L,), jnp.int32),
            pltpu.VMEM((_L,), jnp.int32),
            pltpu.SemaphoreType.DMA,
            pltpu.SemaphoreType.DMA,
        ],
    )(_sc_body)
    kth2 = run(s3)
    kth3 = kth2.reshape(bsz, 1, ensemble)
    out3 = pl.pallas_call(
        _mask_body,
        grid=(bsz,),
        in_specs=[
            pl.BlockSpec((1, 1, ensemble), lambda i: (i, 0, 0)),
            pl.BlockSpec((1, nmax, nmax * ensemble), lambda i: (i, 0, 0)),
        ],
        out_specs=pl.BlockSpec((1, nmax, nmax * ensemble),
                               lambda i: (i, 0, 0)),
        out_shape=jax.ShapeDtypeStruct(s3.shape, jnp.float32),
    )(kth3, s3)
    return out3.reshape(bsz, nmax, nmax, ensemble)


# speedup vs baseline: 1.1579x; 1.0051x over previous
import functools

import jax
import jax.numpy as jnp
import numpy as np
from jax import lax
from jax.experimental import pallas as pl
from jax.experimental.pallas import tpu as pltpu
from jax.experimental.pallas import tpu_sc as plsc

_K = 512
_NC = 2
_NS = 16
_L = 16
_ROW = 1024
_NCHUNK = 4
_CH = 128 // _NCHUNK
_BLOCKS_PER_W = 64 // (_NC * _NS)
_CAPL = 3072

_SIGN = np.int32(-2147483648)


def _ukey(x):
    ui = lax.bitcast_convert_type(x, jnp.int32)
    m = lax.shift_right_arithmetic(ui, np.int32(31))
    return lax.bitcast_convert_type(ui ^ (m | _SIGN), jnp.uint32)


def _sc_body(s_hbm, kth_hbm, buf0, buf1, hist_v, cand_v, kth_v,
             si0, si1):
    lane = lax.iota(jnp.int32, _L)
    ones = jnp.ones((_L,), jnp.int32)
    zeros16 = jnp.zeros((_L,), jnp.int32)
    one_f = jnp.ones((_L,), jnp.float32)
    zero_f = jnp.zeros((_L,), jnp.float32)
    partner = lane ^ 8

    bufs = (buf0, buf1)
    in_sems = (si0, si1)

    wid = lax.axis_index("s") * _NC + lax.axis_index("c")
    blk_b = [wid * _BLOCKS_PER_W + blk for blk in range(_BLOCKS_PER_W)]

    def zero_hist():
        def zero_it(i, _):
            for u in range(8):
                hist_v[pl.ds(i * (_L * 8) + u * _L, _L)] = zeros16
            return 0

        lax.fori_loop(0, 256 // 8, zero_it, 0)

    def hist_sweep(data_v):
        def hist_row(i, _):
            def hist_it(jj, _2):
                for u in range(16):
                    x = data_v[i, pl.ds((jj * 16 + u) * _L, _L)]
                    uk = _ukey(x)
                    bk = lax.shift_right_logical(uk, np.uint32(24))
                    addr = lax.bitcast_convert_type(
                        bk, jnp.int32) * 16 + lane
                    plsc.addupdate_scatter(hist_v, [addr], ones)
                return 0

            lax.fori_loop(0, _ROW // (_L * 16), hist_it, 0)
            return 0

        lax.fori_loop(0, _CH, hist_row, 0)

    def compact_sweep(data_v, b1, off):
        def comp_row(i, off_c):
            def comp_it(jj, off_c2):
                for u in range(16):
                    x = data_v[i, pl.ds((jj * 16 + u) * _L, _L)]
                    uk = _ukey(x)
                    pred = lax.shift_right_logical(
                        uk, np.uint32(24)) == b1
                    addr = jnp.minimum(
                        off_c2, np.int32(_CAPL - 1)) * 16 + lane
                    plsc.store_scatter(
                        cand_v, [addr],
                        lax.bitcast_convert_type(uk, jnp.int32),
                        mask=pred)
                    off_c2 = off_c2 + pred.astype(jnp.int32)
                return off_c2

            return lax.fori_loop(0, _ROW // (_L * 16), comp_it, off_c)

        return lax.fori_loop(0, _CH, comp_row, off)

    def cand_pass(p, prefix, off, nrow4):
        sh_bk = 24 - 8 * p
        sh_pr = 32 - 8 * p

        def cbody(j, _):
            for u in range(4):
                r = j * 4 + u
                v = cand_v[pl.ds(r * _L, _L)]
                uk = lax.bitcast_convert_type(v, jnp.uint32)
                valid = jnp.broadcast_to(r, (_L,)).astype(jnp.int32) < off
                if p == 1:
                    keep = valid
                else:
                    keep = jnp.logical_and(
                        valid,
                        lax.shift_right_logical(
                            uk, np.uint32(sh_pr)) == prefix)
                bk = lax.shift_right_logical(
                    uk, np.uint32(sh_bk)) & np.uint32(0xFF)
                addr = lax.bitcast_convert_type(bk, jnp.int32) * 16 + lane
                plsc.addupdate_scatter(hist_v, [addr], ones, mask=keep)
            return 0

        lax.fori_loop(0, nrow4, cbody, 0)

    def scan_hist(prefix, kk):
        def scan_it(t, carry):
            cum, sel, above, found = carry
            for u in range(2):
                bucket = 255 - (t * 2 + u)
                v = plsc.load_gather(hist_v, [bucket * 16 + lane])
                vsw = plsc.load_gather(hist_v, [bucket * 16 + partner])
                cum_new = cum + v + vsw
                newly = jnp.logical_and(jnp.logical_not(found),
                                        cum_new >= kk)
                bvec = jnp.broadcast_to(bucket, (_L,)).astype(jnp.int32)
                sel = jnp.where(newly, bvec, sel)
                above = jnp.where(newly, cum, above)
                found = jnp.logical_or(found, newly)
                cum = cum_new
            return cum, sel, above, found

        z = jnp.zeros((_L,), jnp.int32)
        _, sel, above, _ = lax.fori_loop(
            0, 128, scan_it, (z, z, z, jnp.zeros((_L,), jnp.bool_)))
        kk = kk - above
        prefix = (prefix << np.uint32(8)) | lax.bitcast_convert_type(
            sel, jnp.uint32)
        return prefix, kk

    sweeps = []
    for blk in range(_BLOCKS_PER_W):
        for c in range(_NCHUNK):
            sweeps.append(("hist", blk, c))
        for c in range(_NCHUNK):
            sweeps.append(("compact", blk, c))

    def src_slice(i):
        _, blk, c = sweeps[i]
        return (blk_b[blk], pl.ds(c * _CH, _CH))

    copies = {}

    def issue_in(i):
        if i >= len(sweeps):
            return
        nb = i % 2
        b, sl = src_slice(i)
        cp = pltpu.make_async_copy(s_hbm.at[b, sl], bufs[nb], in_sems[nb])
        cp.start()
        copies[i] = cp

    zero_hist()
    issue_in(0)
    issue_in(1)

    prefix = jnp.zeros((_L,), jnp.uint32)
    kk = jnp.full((_L,), _K, jnp.int32)
    off = jnp.zeros((_L,), jnp.int32)

    for i, (kind, blk, c) in enumerate(sweeps):
        nb = i % 2
        copies.pop(i).wait()
        if kind == "hist":
            if c == 0:
                prefix = jnp.zeros((_L,), jnp.uint32)
                kk = jnp.full((_L,), _K, jnp.int32)
                off = jnp.zeros((_L,), jnp.int32)
            hist_sweep(bufs[nb])
            issue_in(i + 2)
            if c == _NCHUNK - 1:
                prefix, kk = scan_hist(prefix, kk)
        elif kind == "compact":
            off = compact_sweep(bufs[nb], prefix, off)
            issue_in(i + 2)
            if c == _NCHUNK - 1:
                max_off = lax.reduce_max(off, (0,))
                nrow4 = lax.div(max_off + 3, np.int32(4))
                for p in (1, 2, 3):
                    zero_hist()
                    cand_pass(p, prefix, off, nrow4)
                    prefix, kk = scan_hist(prefix, kk)
                zero_hist()
                kth_v[pl.ds(0, _L)] = lax.bitcast_convert_type(
                    prefix, jnp.int32)
                pltpu.sync_copy(kth_v.at[pl.ds(0, 8)],
                                kth_hbm.at[blk_b[blk]])


def _mask_body(kth_ref, x_ref, o_ref):
    x = x_ref[...]
    ui = lax.bitcast_convert_type(x, jnp.int32)
    m = lax.shift_right_arithmetic(ui, np.int32(31))
    uk = lax.bitcast_convert_type(ui ^ (m | _SIGN), jnp.uint32)
    idx8 = lax.broadcasted_iota(jnp.int32, (1, 1, _ROW), 2) % 8
    t = jnp.zeros((1, 1, _ROW), jnp.uint32)
    for e in range(8):
        te = lax.bitcast_convert_type(kth_ref[0, 0, e], jnp.uint32)
        t = jnp.where(idx8 == e, te, t)
    o_ref[...] = jnp.where(uk >= t, np.float32(1.0), np.float32(0.0))


@jax.jit
def kernel(scores):
    bsz, nmax, _, ensemble = scores.shape
    s3 = scores.reshape(bsz, nmax, nmax * ensemble)
    run = functools.partial(
        pl.kernel,
        mesh=plsc.VectorSubcoreMesh(core_axis_name="c",
                                    subcore_axis_name="s"),
        out_type=jax.ShapeDtypeStruct((bsz, ensemble), jnp.int32),
        compiler_params=pltpu.CompilerParams(
            needs_layout_passes=False, use_tc_tiling_on_sc=False),
        scratch_types=[
            pltpu.VMEM((_CH, _ROW), jnp.float32),
            pltpu.VMEM((_CH, _ROW), jnp.float32),
            pltpu.VMEM((256 * _L,), jnp.int32),
            pltpu.VMEM((_CAPL * _L,), jnp.int32),
            pltpu.VMEM((_L,), jnp.int32),
            pltpu.SemaphoreType.DMA,
            pltpu.SemaphoreType.DMA,
        ],
    )(_sc_body)
    kth2 = run(s3)
    kth3 = kth2.reshape(bsz, 1, ensemble)
    out3 = pl.pallas_call(
        _mask_body,
        grid=(bsz,),
        in_specs=[
            pl.BlockSpec((1, 1, ensemble), lambda i: (i, 0, 0)),
            pl.BlockSpec((1, nmax, nmax * ensemble), lambda i: (i, 0, 0)),
        ],
        out_specs=pl.BlockSpec((1, nmax, nmax * ensemble),
                               lambda i: (i, 0, 0)),
        out_shape=jax.ShapeDtypeStruct(s3.shape, jnp.float32),
    )(kth3, s3)
    return out3.reshape(bsz, nmax, nmax, ensemble)
